# baseline (speedup 1.0000x reference)
import functools

import jax
import jax.numpy as jnp
from jax import lax
from jax.experimental import pallas as pl
from jax.experimental.pallas import tpu as pltpu
from jax.experimental.pallas import tpu_sc as plsc

_NC = 2
_NS = 16
_NW = _NC * _NS
_L = 16
_D = 128


def _table_kernel(w_ref, kind_ref, strpc_ref, feat_ref, bias_ref, tab_ref):
    w = w_ref[:]
    norms = jnp.sqrt(jnp.sum(w * w, axis=1, keepdims=True))
    scale = jnp.where(norms > 1.0, 1.0 / (norms + 1e-7), jnp.ones_like(norms))
    wr = w * scale
    wr32 = jnp.concatenate([wr, jnp.zeros((7, 32), jnp.float32)], axis=0)

    feat = feat_ref[:]
    c_row = bias_ref[:]
    for k in range(6):
        c_row = c_row + lax.dot_general(
            kind_ref[k:k + 1, :], feat[:, 32 * k:32 * (k + 1)],
            (((1,), (1,)), ((), ())), preferred_element_type=jnp.float32)
    v_row = lax.dot_general(
        strpc_ref[:], feat[:, 0:32], (((1,), (1,)), ((), ())),
        preferred_element_type=jnp.float32) * (1.0 / 99.0)

    row_ids = lax.broadcasted_iota(jnp.int32, (32, 128), 0)
    for k in range(6):
        blk = feat[:, 32 * k:32 * (k + 1)]
        tk = lax.dot_general(wr32, blk, (((1,), (1,)), ((), ())),
                             preferred_element_type=jnp.float32)
        if k == 0:
            tk = jnp.where(row_ids < 25, tk + c_row, tk)
            tk = jnp.where(row_ids == 25, v_row, tk)
        tab_ref[32 * k:32 * (k + 1), :] = tk


def _build_table(stat_weight, kind_weight, strpc_weight, feat_weight, feat_bias):
    return pl.pallas_call(
        _table_kernel,
        out_shape=jax.ShapeDtypeStruct((192, 128), jnp.float32),
    )(stat_weight, kind_weight, strpc_weight.T, feat_weight,
      feat_bias.reshape(1, 128))


def _make_sc_lookup(B):
    assert B % (8 * _NW) == 0
    b_per_w = B // _NW
    mesh = plsc.VectorSubcoreMesh(core_axis_name="c", subcore_axis_name="s",
                                  num_cores=_NC, num_subcores=_NS)

    @functools.partial(
        pl.kernel,
        out_type=jax.ShapeDtypeStruct((B, _D), jnp.float32),
        mesh=mesh,
        compiler_params=pltpu.CompilerParams(needs_layout_passes=False),
        scratch_types=[
            pltpu.VMEM((192, _D), jnp.float32),
            pltpu.VMEM((b_per_w, _D), jnp.float32),
            pltpu.VMEM((6, b_per_w), jnp.int32),
            pltpu.VMEM((b_per_w,), jnp.float32),
            pltpu.SemaphoreType.DMA,
            pltpu.SemaphoreType.DMA,
        ],
    )
    def sc_lookup(tab_hbm, i0, i1, i2, i3, i4, i5, sp_hbm, out_hbm,
                  tab_v, out_v, idx_s, sp_s, in_sem, out_sem):
        wid = lax.axis_index("s") * _NC + lax.axis_index("c")
        base = wid * b_per_w
        descs = [pltpu.async_copy(tab_hbm, tab_v, in_sem)]
        for k, ik in enumerate((i0, i1, i2, i3, i4, i5)):
            descs.append(pltpu.async_copy(ik.at[pl.ds(base, b_per_w)],
                                          idx_s.at[k], in_sem))
        descs.append(pltpu.async_copy(sp_hbm.at[pl.ds(base, b_per_w)],
                                      sp_s, in_sem))
        for d in descs:
            d.wait()

        vregs = [tab_v[25, pl.ds(16 * j, _L)] for j in range(8)]

        @plsc.parallel_loop(0, b_per_w // _L)
        def grp_body(gi):
            goff = gi * _L
            ivecs = [idx_s[k, pl.ds(goff, _L)] + 32 * k for k in range(6)]
            spvec = sp_s[pl.ds(goff, _L)]

            @plsc.parallel_loop(0, _L, unroll=2)
            def lane_body(l):
                lsplat = jnp.broadcast_to(l, (_L,)).astype(jnp.int32)
                spb = spvec.at[lsplat].get(mode="promise_in_bounds")
                accs = [spb * vregs[j] for j in range(8)]
                for k in range(6):
                    row = ivecs[k].at[lsplat].get(
                        mode="promise_in_bounds")[0]
                    for j in range(8):
                        accs[j] = accs[j] + tab_v[row, pl.ds(16 * j, _L)]
                for j in range(8):
                    out_v[goff + l, pl.ds(16 * j, _L)] = accs[j]

            pltpu.async_copy(out_v.at[pl.ds(goff, _L)],
                             out_hbm.at[pl.ds(base + goff, _L)],
                             out_sem)

        pltpu.make_async_copy(
            out_v, out_hbm.at[pl.ds(base, b_per_w)], out_sem).wait()

    return sc_lookup


def kernel(str, dex, con, int, wis, cha, strength_percentage,
           stat_weight, kind_weight, strpc_weight, feat_weight, feat_bias):
    B = str.shape[0]
    tab = _build_table(stat_weight, kind_weight, strpc_weight, feat_weight,
                       feat_bias)
    return _make_sc_lookup(B)(tab, str, dex, con, int, wis, cha,
                              strength_percentage)


# speedup vs baseline: 1.2553x; 1.2553x over previous
import functools

import jax
import jax.numpy as jnp
from jax import lax
from jax.experimental import pallas as pl
from jax.experimental.pallas import tpu as pltpu
from jax.experimental.pallas import tpu_sc as plsc

_NC = 2
_NS = 16
_NW = _NC * _NS
_L = 16
_D = 128


def _table_kernel(w_ref, kind_ref, strpc_ref, feat_ref, bias_ref, tab_ref):
    w = w_ref[:]
    norms = jnp.sqrt(jnp.sum(w * w, axis=1, keepdims=True))
    scale = jnp.where(norms > 1.0, 1.0 / (norms + 1e-7), jnp.ones_like(norms))
    wr = w * scale
    wr32 = jnp.concatenate([wr, jnp.zeros((7, 32), jnp.float32)], axis=0)

    feat = feat_ref[:]
    c_row = bias_ref[:]
    for k in range(6):
        c_row = c_row + lax.dot_general(
            kind_ref[k:k + 1, :], feat[:, 32 * k:32 * (k + 1)],
            (((1,), (1,)), ((), ())), preferred_element_type=jnp.float32)
    v_row = lax.dot_general(
        strpc_ref[:], feat[:, 0:32], (((0,), (1,)), ((), ())),
        preferred_element_type=jnp.float32)
    v_row = v_row * (1.0 / 99.0)

    row_ids = lax.broadcasted_iota(jnp.int32, (32, 128), 0)
    for k in range(6):
        blk = feat[:, 32 * k:32 * (k + 1)]
        tk = lax.dot_general(wr32, blk, (((1,), (1,)), ((), ())),
                             preferred_element_type=jnp.float32)
        if k == 0:
            tk = jnp.where(row_ids < 25, tk + c_row, tk)
            tk = jnp.where(row_ids == 25, v_row, tk)
        tab_ref[32 * k:32 * (k + 1), :] = tk


def _build_table(stat_weight, kind_weight, strpc_weight, feat_weight, feat_bias):
    return pl.pallas_call(
        _table_kernel,
        out_shape=jax.ShapeDtypeStruct((192, 128), jnp.float32),
    )(stat_weight, kind_weight, strpc_weight, feat_weight, feat_bias)


def _make_sc_lookup(B):
    assert B % (8 * _NW) == 0
    b_per_w = B // _NW
    mesh = plsc.VectorSubcoreMesh(core_axis_name="c", subcore_axis_name="s",
                                  num_cores=_NC, num_subcores=_NS)

    @functools.partial(
        pl.kernel,
        out_type=jax.ShapeDtypeStruct((B * _D,), jnp.float32),
        mesh=mesh,
        compiler_params=pltpu.CompilerParams(needs_layout_passes=False),
        scratch_types=[
            pltpu.VMEM((192 * _D,), jnp.float32),
            pltpu.VMEM((b_per_w * _D,), jnp.float32),
            pltpu.VMEM((6, b_per_w), jnp.int32),
            pltpu.VMEM((b_per_w,), jnp.float32),
            pltpu.SemaphoreType.DMA,
            pltpu.SemaphoreType.DMA,
        ],
    )
    def sc_lookup(tab_hbm, i0, i1, i2, i3, i4, i5, sp_hbm, out_hbm,
                  tab_v, out_v, idx_s, sp_s, in_sem, out_sem):
        wid = lax.axis_index("s") * _NC + lax.axis_index("c")
        base = wid * b_per_w
        descs = [pltpu.async_copy(tab_hbm, tab_v, in_sem)]
        for k, ik in enumerate((i0, i1, i2, i3, i4, i5)):
            descs.append(pltpu.async_copy(ik.at[pl.ds(base, b_per_w)],
                                          idx_s.at[k], in_sem))
        descs.append(pltpu.async_copy(sp_hbm.at[pl.ds(base, b_per_w)],
                                      sp_s, in_sem))
        for d in descs:
            d.wait()

        vregs = [tab_v[pl.ds(25 * _D + 16 * j, _L)] for j in range(8)]

        @plsc.parallel_loop(0, b_per_w // _L)
        def grp_body(gi):
            goff = gi * _L
            ivecs = [(idx_s[k, pl.ds(goff, _L)] + 32 * k) * _D
                     for k in range(6)]
            spvec = sp_s[pl.ds(goff, _L)]

            @plsc.parallel_loop(0, _L)
            def lane_body(l):
                lsplat = jnp.broadcast_to(l, (_L,)).astype(jnp.int32)
                spb = spvec.at[lsplat].get(mode="promise_in_bounds")
                accs = [spb * vregs[j] for j in range(8)]
                for k in range(6):
                    rbase = ivecs[k].at[lsplat].get(
                        mode="promise_in_bounds")[0]
                    for j in range(8):
                        accs[j] = accs[j] + tab_v[pl.ds(rbase + 16 * j, _L)]
                obase = (goff + l) * _D
                for j in range(8):
                    out_v[pl.ds(obase + 16 * j, _L)] = accs[j]

            pltpu.async_copy(out_v.at[pl.ds(goff * _D, _L * _D)],
                             out_hbm.at[pl.ds((base + goff) * _D, _L * _D)],
                             out_sem)

        pltpu.make_async_copy(
            out_v, out_hbm.at[pl.ds(base * _D, b_per_w * _D)], out_sem).wait()

    return sc_lookup


def kernel(str, dex, con, int, wis, cha, strength_percentage,
           stat_weight, kind_weight, strpc_weight, feat_weight, feat_bias):
    B = str.shape[0]
    tab = _build_table(stat_weight, kind_weight, strpc_weight, feat_weight,
                       feat_bias)
    flat = _make_sc_lookup(B)(tab.reshape(-1), str, dex, con, int, wis, cha,
                              strength_percentage)
    return flat.reshape(B, _D)


# speedup vs baseline: 1.2569x; 1.0013x over previous
import functools

import jax
import jax.numpy as jnp
from jax import lax
from jax.experimental import pallas as pl
from jax.experimental.pallas import tpu as pltpu
from jax.experimental.pallas import tpu_sc as plsc

_NC = 2
_NS = 16
_NW = _NC * _NS
_L = 16
_D = 128


def _table_kernel(w_hbm, kind_hbm, strpc_hbm, feat_hbm, bias_hbm, tab_hbm,
                  w_ref, kind_ref, strpc_ref, feat_ref, bias_ref, tab_ref,
                  sem):
    descs = [
        pltpu.make_async_copy(w_hbm, w_ref, sem),
        pltpu.make_async_copy(kind_hbm, kind_ref, sem),
        pltpu.make_async_copy(strpc_hbm, strpc_ref, sem),
        pltpu.make_async_copy(feat_hbm, feat_ref, sem),
        pltpu.make_async_copy(bias_hbm, bias_ref, sem),
    ]
    for d in descs:
        d.start()
    for d in descs:
        d.wait()
    w = w_ref[:]
    norms = jnp.sqrt(jnp.sum(w * w, axis=1, keepdims=True))
    scale = jnp.where(norms > 1.0, 1.0 / (norms + 1e-7), jnp.ones_like(norms))
    wr = w * scale
    wr32 = jnp.concatenate([wr, jnp.zeros((7, 32), jnp.float32)], axis=0)

    feat = feat_ref[:]
    c_row = bias_ref[:]
    for k in range(6):
        c_row = c_row + lax.dot_general(
            kind_ref[k:k + 1, :], feat[:, 32 * k:32 * (k + 1)],
            (((1,), (1,)), ((), ())), preferred_element_type=jnp.float32)
    v_row = lax.dot_general(
        strpc_ref[:], feat[:, 0:32], (((0,), (1,)), ((), ())),
        preferred_element_type=jnp.float32)
    v_row = v_row * (1.0 / 99.0)

    row_ids = lax.broadcasted_iota(jnp.int32, (32, 128), 0)
    for k in range(6):
        blk = feat[:, 32 * k:32 * (k + 1)]
        tk = lax.dot_general(wr32, blk, (((1,), (1,)), ((), ())),
                             preferred_element_type=jnp.float32)
        if k == 0:
            tk = jnp.where(row_ids < 25, tk + c_row, tk)
            tk = jnp.where(row_ids == 25, v_row, tk)
        tab_ref[32 * k:32 * (k + 1), :] = tk
    pltpu.make_async_copy(tab_ref, tab_hbm, sem).start()
    pltpu.make_async_copy(tab_ref, tab_hbm, sem).wait()


def _build_table(stat_weight, kind_weight, strpc_weight, feat_weight, feat_bias):
    any_spec = pl.BlockSpec(memory_space=pl.ANY)
    return pl.pallas_call(
        _table_kernel,
        out_shape=jax.ShapeDtypeStruct((192, 128), jnp.float32),
        in_specs=[any_spec] * 5,
        out_specs=any_spec,
        scratch_shapes=[
            pltpu.VMEM((25, 32), jnp.float32),
            pltpu.VMEM((6, 32), jnp.float32),
            pltpu.VMEM((32, 1), jnp.float32),
            pltpu.VMEM((128, 192), jnp.float32),
            pltpu.VMEM((128,), jnp.float32),
            pltpu.VMEM((192, 128), jnp.float32),
            pltpu.SemaphoreType.DMA,
        ],
    )(stat_weight, kind_weight, strpc_weight, feat_weight, feat_bias)


def _make_sc_lookup(B):
    assert B % (8 * _NW) == 0
    b_per_w = B // _NW
    mesh = plsc.VectorSubcoreMesh(core_axis_name="c", subcore_axis_name="s",
                                  num_cores=_NC, num_subcores=_NS)

    @functools.partial(
        pl.kernel,
        out_type=jax.ShapeDtypeStruct((B * _D,), jnp.float32),
        mesh=mesh,
        compiler_params=pltpu.CompilerParams(needs_layout_passes=False),
        scratch_types=[
            pltpu.VMEM((192 * _D,), jnp.float32),
            pltpu.VMEM((b_per_w * _D,), jnp.float32),
            pltpu.VMEM((6, b_per_w), jnp.int32),
            pltpu.VMEM((b_per_w,), jnp.float32),
            pltpu.SemaphoreType.DMA,
            pltpu.SemaphoreType.DMA,
        ],
    )
    def sc_lookup(tab_hbm, i0, i1, i2, i3, i4, i5, sp_hbm, out_hbm,
                  tab_v, out_v, idx_s, sp_s, in_sem, out_sem):
        wid = lax.axis_index("s") * _NC + lax.axis_index("c")
        base = wid * b_per_w
        descs = [pltpu.async_copy(tab_hbm, tab_v, in_sem)]
        for k, ik in enumerate((i0, i1, i2, i3, i4, i5)):
            descs.append(pltpu.async_copy(ik.at[pl.ds(base, b_per_w)],
                                          idx_s.at[k], in_sem))
        descs.append(pltpu.async_copy(sp_hbm.at[pl.ds(base, b_per_w)],
                                      sp_s, in_sem))
        for d in descs:
            d.wait()

        vregs = [tab_v[pl.ds(25 * _D + 16 * j, _L)] for j in range(8)]

        @plsc.parallel_loop(0, b_per_w // _L)
        def grp_body(gi):
            goff = gi * _L
            ivecs = [(idx_s[k, pl.ds(goff, _L)] + 32 * k) * _D
                     for k in range(6)]
            spvec = sp_s[pl.ds(goff, _L)]

            @plsc.parallel_loop(0, _L)
            def lane_body(l):
                lsplat = jnp.broadcast_to(l, (_L,)).astype(jnp.int32)
                spb = spvec.at[lsplat].get(mode="promise_in_bounds")
                accs = [spb * vregs[j] for j in range(8)]
                for k in range(6):
                    rbase = ivecs[k].at[lsplat].get(
                        mode="promise_in_bounds")[0]
                    for j in range(8):
                        accs[j] = accs[j] + tab_v[pl.ds(rbase + 16 * j, _L)]
                obase = (goff + l) * _D
                for j in range(8):
                    out_v[pl.ds(obase + 16 * j, _L)] = accs[j]

            pltpu.async_copy(out_v.at[pl.ds(goff * _D, _L * _D)],
                             out_hbm.at[pl.ds((base + goff) * _D, _L * _D)],
                             out_sem)

        pltpu.make_async_copy(
            out_v, out_hbm.at[pl.ds(base * _D, b_per_w * _D)], out_sem).wait()

    return sc_lookup


def kernel(str, dex, con, int, wis, cha, strength_percentage,
           stat_weight, kind_weight, strpc_weight, feat_weight, feat_bias):
    B = str.shape[0]
    tab = _build_table(stat_weight, kind_weight, strpc_weight, feat_weight,
                       feat_bias)
    flat = _make_sc_lookup(B)(tab.reshape(-1), str, dex, con, int, wis, cha,
                              strength_percentage)
    return flat.reshape(B, _D)


# speedup vs baseline: 1.3577x; 1.0802x over previous
import functools

import jax
import jax.numpy as jnp
from jax import lax
from jax.experimental import pallas as pl
from jax.experimental.pallas import tpu as pltpu
from jax.experimental.pallas import tpu_sc as plsc

_NC = 2
_NS = 16
_NW = _NC * _NS
_L = 16
_D = 128


def _table_kernel(w_ref, kind_ref, strpc_ref, feat_ref, bias_ref, tab_ref):
    w = w_ref[:]
    norms = jnp.sqrt(jnp.sum(w * w, axis=1, keepdims=True))
    scale = jnp.where(norms > 1.0, 1.0 / (norms + 1e-7), jnp.ones_like(norms))
    wr = w * scale
    wr32 = jnp.concatenate([wr, jnp.zeros((7, 32), jnp.float32)], axis=0)

    featT = feat_ref[:]
    c_row = bias_ref[:]
    for k in range(6):
        c_row = c_row + lax.dot_general(
            kind_ref[k:k + 1, :], featT[32 * k:32 * (k + 1), :],
            (((1,), (0,)), ((), ())), preferred_element_type=jnp.float32)
    v_row = lax.dot_general(
        strpc_ref[:], featT[0:32, :], (((1,), (0,)), ((), ())),
        preferred_element_type=jnp.float32)
    v_row = v_row * (1.0 / 99.0)

    row_ids = lax.broadcasted_iota(jnp.int32, (32, 128), 0)
    for k in range(6):
        blkT = featT[32 * k:32 * (k + 1), :]
        tk = lax.dot_general(wr32, blkT, (((1,), (0,)), ((), ())),
                             preferred_element_type=jnp.float32)
        if k == 0:
            tk = jnp.where(row_ids < 25, tk + c_row, tk)
            tk = jnp.where(row_ids == 25, v_row, tk)
        tab_ref[32 * k:32 * (k + 1), :] = tk


def _build_table(stat_weight, kind_weight, strpc_weight, feat_weight, feat_bias):
    return pl.pallas_call(
        _table_kernel,
        out_shape=jax.ShapeDtypeStruct((192, 128), jnp.float32),
    )(stat_weight, kind_weight, strpc_weight.T, feat_weight.T, feat_bias)


def _make_sc_lookup(B):
    assert B % (8 * _NW) == 0
    b_per_w = B // _NW
    mesh = plsc.VectorSubcoreMesh(core_axis_name="c", subcore_axis_name="s",
                                  num_cores=_NC, num_subcores=_NS)

    @functools.partial(
        pl.kernel,
        out_type=jax.ShapeDtypeStruct((B * _D,), jnp.float32),
        mesh=mesh,
        compiler_params=pltpu.CompilerParams(needs_layout_passes=False),
        scratch_types=[
            pltpu.VMEM((192 * _D,), jnp.float32),
            pltpu.VMEM((b_per_w * _D,), jnp.float32),
            pltpu.VMEM((6, b_per_w), jnp.int32),
            pltpu.VMEM((b_per_w,), jnp.float32),
            pltpu.SemaphoreType.DMA,
            pltpu.SemaphoreType.DMA,
        ],
    )
    def sc_lookup(tab_hbm, i0, i1, i2, i3, i4, i5, sp_hbm, out_hbm,
                  tab_v, out_v, idx_s, sp_s, in_sem, out_sem):
        wid = lax.axis_index("s") * _NC + lax.axis_index("c")
        base = wid * b_per_w
        descs = [pltpu.async_copy(tab_hbm, tab_v, in_sem)]
        for k, ik in enumerate((i0, i1, i2, i3, i4, i5)):
            descs.append(pltpu.async_copy(ik.at[pl.ds(base, b_per_w)],
                                          idx_s.at[k], in_sem))
        descs.append(pltpu.async_copy(sp_hbm.at[pl.ds(base, b_per_w)],
                                      sp_s, in_sem))
        for d in descs:
            d.wait()

        vregs = [tab_v[pl.ds(25 * _D + 16 * j, _L)] for j in range(8)]

        @plsc.parallel_loop(0, b_per_w // _L)
        def grp_body(gi):
            goff = gi * _L
            ivecs = [(idx_s[k, pl.ds(goff, _L)] + 32 * k) * _D
                     for k in range(6)]
            spvec = sp_s[pl.ds(goff, _L)]

            @plsc.parallel_loop(0, _L)
            def lane_body(l):
                lsplat = jnp.broadcast_to(l, (_L,)).astype(jnp.int32)
                spb = spvec.at[lsplat].get(mode="promise_in_bounds")
                accs = [spb * vregs[j] for j in range(8)]
                for k in range(6):
                    rbase = ivecs[k].at[lsplat].get(
                        mode="promise_in_bounds")[0]
                    for j in range(8):
                        accs[j] = accs[j] + tab_v[pl.ds(rbase + 16 * j, _L)]
                obase = (goff + l) * _D
                for j in range(8):
                    out_v[pl.ds(obase + 16 * j, _L)] = accs[j]

            pltpu.async_copy(out_v.at[pl.ds(goff * _D, _L * _D)],
                             out_hbm.at[pl.ds((base + goff) * _D, _L * _D)],
                             out_sem)

        pltpu.make_async_copy(
            out_v, out_hbm.at[pl.ds(base * _D, b_per_w * _D)], out_sem).wait()

    return sc_lookup


def kernel(str, dex, con, int, wis, cha, strength_percentage,
           stat_weight, kind_weight, strpc_weight, feat_weight, feat_bias):
    B = str.shape[0]
    tab = _build_table(stat_weight, kind_weight, strpc_weight, feat_weight,
                       feat_bias)
    flat = _make_sc_lookup(B)(tab.reshape(-1), str, dex, con, int, wis, cha,
                              strength_percentage)
    return flat.reshape(B, _D)
